# baseline (speedup 1.0000x reference)
import functools

import jax
import jax.numpy as jnp
from jax import lax
from jax.experimental import pallas as pl
from jax.experimental.pallas import tpu as pltpu
from jax.experimental.pallas import tpu_sc as plsc

N_NODES = 10000
N_EDGES = 320000
D = 128
_BN = 2000

_NW = 32
_NPW = 313
_NPAD = _NW * _NPW
_C = 4000
_NCHUNK = N_EDGES // _C
_RB = 256
_G16 = 16


def _seg_max_body(m_hbm, src_hbm, dst_hbm, out_hbm,
                  srcv, dstv, fsrc, fdst, rows, acc, sem):
    cid = lax.axis_index("c")
    sid = lax.axis_index("s")
    wid = sid * 2 + cid
    lo = wid * _NPW
    hi = lo + _NPW
    zero16i = jnp.zeros((16,), jnp.int32)
    zero16f = jnp.zeros((16,), jnp.float32)
    sent16 = jnp.full((16,), _NPW, jnp.int32)

    def initacc(i, c):
        acc[pl.ds(i * 16, 16)] = zero16f
        return c
    lax.fori_loop(0, (_NPW + 1) * D // 16, initacc, 0)

    def initf(i, c):
        fsrc[pl.ds(i * 16, 16)] = zero16i
        return c
    lax.fori_loop(0, (_C + 16) // 16, initf, 0)

    def chunk_body(ec, carry):
        ebase = ec * _C
        pltpu.sync_copy(src_hbm.at[pl.ds(ebase, _C)], srcv)
        pltpu.sync_copy(dst_hbm.at[pl.ds(ebase, _C)], dstv)

        def fbody(v, cur):
            d16 = dstv[pl.ds(v * 16, 16)]
            s16 = srcv[pl.ds(v * 16, 16)]
            msk = (d16 >= lo) & (d16 < hi)
            plsc.store_compressed(fsrc.at[pl.ds(cur, 16)], s16, mask=msk)
            plsc.store_compressed(fdst.at[pl.ds(cur, 16)], d16 - lo, mask=msk)
            return cur + plsc.all_reduce_population_count(msk)[0]
        nm = lax.fori_loop(0, _C // 16, fbody, 0)

        fdst[pl.ds(nm, 16)] = sent16

        nm16 = ((nm + 15) // 16) * 16
        nb = (nm16 + _RB - 1) // _RB

        def bbody(b, c):
            boff = b * _RB
            for t in range(_RB // 128):
                @pl.when(boff + t * 128 < nm16)
                def _fire(t=t):
                    pltpu.make_async_copy(
                        m_hbm.at[fsrc.at[pl.ds(boff + t * 128, 128)]],
                        rows.at[pl.ds(t * 128, 128)], sem).start()
            for t in range(_RB // 128):
                @pl.when(boff + t * 128 < nm16)
                def _drain(t=t):
                    pltpu.make_async_copy(
                        m_hbm.at[fsrc.at[pl.ds(boff + t * 128, 128)]],
                        rows.at[pl.ds(t * 128, 128)], sem).wait()

            ng = jnp.minimum((nm16 - boff + 15) // 16, _RB // 16)

            def gbody(g, cc):
                d16 = fdst[pl.ds(boff + g * 16, 16)]
                for j in range(16):
                    base = d16[j] * D
                    r = g * 16 + j
                    for q in range(D // 16):
                        a = acc[pl.ds(base + q * 16, 16)]
                        v = rows[r, pl.ds(q * 16, 16)]
                        acc[pl.ds(base + q * 16, 16)] = jnp.maximum(a, v)
                return cc
            lax.fori_loop(0, ng, gbody, 0)
            return c
        lax.fori_loop(0, nb, bbody, 0)
        return carry
    lax.fori_loop(0, _NCHUNK, chunk_body, 0)

    pltpu.sync_copy(acc.at[pl.ds(0, _NPW * D)],
                    out_hbm.at[pl.ds(wid * _NPW * D, _NPW * D)])


@functools.partial(
    pl.kernel,
    mesh=plsc.VectorSubcoreMesh(core_axis_name="c", subcore_axis_name="s"),
    out_type=jax.ShapeDtypeStruct((_NPAD * D,), jnp.float32),
    scratch_types=[
        pltpu.VMEM((_C,), jnp.int32),
        pltpu.VMEM((_C,), jnp.int32),
        pltpu.VMEM((_C + 16,), jnp.int32),
        pltpu.VMEM((_C + 16,), jnp.int32),
        pltpu.VMEM((_RB, D), jnp.float32),
        pltpu.VMEM(((_NPW + 1) * D,), jnp.float32),
        pltpu.SemaphoreType.DMA,
    ],
    compiler_params=pltpu.CompilerParams(needs_layout_passes=False),
)
def _seg_max_sc(m_hbm, src_hbm, dst_hbm, out_hbm,
                srcv, dstv, fsrc, fdst, rows, acc, sem):
    _seg_max_body(m_hbm, src_hbm, dst_hbm, out_hbm,
                  srcv, dstv, fsrc, fdst, rows, acc, sem)


def _segment_max(m, src, dst):
    agg = _seg_max_sc(m, src, dst)
    return agg.reshape(_NPAD, D)[:N_NODES]


def _pool_kernel(x_ref, wp_ref, bp_ref, m_ref):
    m = jnp.dot(x_ref[...], wp_ref[...], preferred_element_type=jnp.float32) + bp_ref[...]
    m_ref[...] = jnp.maximum(m, 0.0)


def _pool_mlp(x, Wpool, bpool):
    n = x.shape[0]
    return pl.pallas_call(
        _pool_kernel,
        grid=(n // _BN,),
        in_specs=[
            pl.BlockSpec((_BN, D), lambda i: (i, 0)),
            pl.BlockSpec((D, D), lambda i: (0, 0)),
            pl.BlockSpec((1, D), lambda i: (0, 0)),
        ],
        out_specs=pl.BlockSpec((_BN, D), lambda i: (i, 0)),
        out_shape=jax.ShapeDtypeStruct((n, D), jnp.float32),
    )(x, Wpool.T, bpool.reshape(1, D))


def _combine_kernel(x_ref, a_ref, ws_ref, wn_ref, b_ref, wp_ref, bp_ref,
                    h_ref, m_ref, hmax_ref):
    i = pl.program_id(0)
    h = (jnp.dot(x_ref[...], ws_ref[...], preferred_element_type=jnp.float32)
         + jnp.dot(a_ref[...], wn_ref[...], preferred_element_type=jnp.float32)
         + b_ref[...])
    h = jnp.maximum(h, 0.0)
    h_ref[...] = h
    m = jnp.dot(h, wp_ref[...], preferred_element_type=jnp.float32) + bp_ref[...]
    m_ref[...] = jnp.maximum(m, 0.0)
    blkmax = jnp.max(h, axis=0, keepdims=True)
    @pl.when(i == 0)
    def _():
        hmax_ref[...] = blkmax
    @pl.when(i > 0)
    def _():
        hmax_ref[...] = jnp.maximum(hmax_ref[...], blkmax)


def _sage_combine(x, agg, Wself, Wneigh, bias, Wpool_next, bpool_next):
    n = x.shape[0]
    h, m, hmax = pl.pallas_call(
        _combine_kernel,
        grid=(n // _BN,),
        in_specs=[
            pl.BlockSpec((_BN, D), lambda i: (i, 0)),
            pl.BlockSpec((_BN, D), lambda i: (i, 0)),
            pl.BlockSpec((D, D), lambda i: (0, 0)),
            pl.BlockSpec((D, D), lambda i: (0, 0)),
            pl.BlockSpec((1, D), lambda i: (0, 0)),
            pl.BlockSpec((D, D), lambda i: (0, 0)),
            pl.BlockSpec((1, D), lambda i: (0, 0)),
        ],
        out_specs=[
            pl.BlockSpec((_BN, D), lambda i: (i, 0)),
            pl.BlockSpec((_BN, D), lambda i: (i, 0)),
            pl.BlockSpec((1, D), lambda i: (0, 0)),
        ],
        out_shape=[
            jax.ShapeDtypeStruct((n, D), jnp.float32),
            jax.ShapeDtypeStruct((n, D), jnp.float32),
            jax.ShapeDtypeStruct((1, D), jnp.float32),
        ],
    )(x, agg, Wself.T, Wneigh.T, bias.reshape(1, D), Wpool_next.T,
      bpool_next.reshape(1, D))
    return h, m, hmax


def kernel(inputs, edge_index, states, actions, Wpool1, bpool1, Wneigh1,
           Wself1, bias1, Wpool2, bpool2, Wneigh2, Wself2, bias2, fc2_W,
           fc2_b, fc3_W, fc3_b):
    src = edge_index[0]
    dst = edge_index[1]

    m1 = _pool_mlp(inputs, Wpool1, bpool1)
    agg1 = _segment_max(m1, src, dst)
    h1, m2, _ = _sage_combine(inputs, agg1, Wself1, Wneigh1, bias1,
                              Wpool2, bpool2)
    agg2 = _segment_max(m2, src, dst)
    h2, _, graph_aggvector = _sage_combine(h1, agg2, Wself2, Wneigh2, bias2,
                                           Wpool2, bpool2)

    states_vector = jnp.take(h2, states, axis=0)
    actions_vector = jnp.take(h2, actions, axis=0)
    states_aggvector = jnp.max(states_vector, axis=0, keepdims=True)
    hc = jnp.concatenate([graph_aggvector, states_aggvector, actions_vector],
                         axis=1)
    out = jax.nn.relu(hc @ fc2_W.T + fc2_b)
    out = out @ fc3_W.T + fc3_b
    return out
